# baseline (speedup 1.0000x reference)
import functools

import jax
import jax.numpy as jnp
from jax import lax
from jax.experimental import pallas as pl
from jax.experimental.pallas import tpu as pltpu
from jax.experimental.pallas import tpu_sc as plsc

N = 10000
E = 320000
D = 128
H = 32
C = 16

NC = 2
NS = 16
NW = NC * NS

SLICE1 = 640
NPAD1 = NS * SLICE1
EPT1 = E // NW
CH1 = 1000
NCH1 = EPT1 // CH1

EPS = E // NS
CH2 = 2000
NCH2 = EPS // CH2
LRUN = 320
CSRCAP = EPS + NS * 8
BINROW = CSRCAP + 1024 + 32
CHB = 1024

_mesh = plsc.VectorSubcoreMesh(core_axis_name="c", subcore_axis_name="s")
_sc_params = pltpu.CompilerParams(use_tc_tiling_on_sc=False,
                                  needs_layout_passes=False)


def _fill_f32(ref, n_vecs, val):
    v = jnp.full((16,), val, jnp.float32)

    def body(i, carry):
        ref[pl.ds(i * 16, 16)] = v
        return carry

    lax.fori_loop(0, n_vecs, body, 0)


def _zero_rows(ref, rows):
    z = jnp.zeros((16,), jnp.float32)

    def body(i, carry):
        r = i >> 1
        cs = (i & 1) * 16
        ref[r, pl.ds(cs, 16)] = z
        return carry

    lax.fori_loop(0, rows * 2, body, 0)


def _sc1_body(y_hbm, src_hbm, dst_hbm, acc_out, cnt_out,
              idx_v, rows_v, ones_v, zbuf_v, zcnt_v, acc_sh, cnt_sh, sem):
    c = lax.axis_index("c")
    s = lax.axis_index("s")
    wid = c * NS + s

    _zero_rows(zbuf_v, SLICE1)
    _fill_f32(zcnt_v, SLICE1 // 16, 0.0)
    _fill_f32(ones_v, CH1 // 16 + 1, 1.0)
    nbase = s * SLICE1
    pltpu.sync_copy(zbuf_v, acc_sh.at[pl.ds(nbase, SLICE1)])
    pltpu.sync_copy(zcnt_v.at[pl.ds(0, SLICE1)], cnt_sh.at[pl.ds(nbase, SLICE1)])
    plsc.subcore_barrier()

    ebase = wid * EPT1

    def chunk(k, carry):
        off = pl.multiple_of(ebase + k * CH1, 8)
        pltpu.sync_copy(src_hbm.at[pl.ds(off, CH1)], idx_v)
        pltpu.async_copy(y_hbm.at[idx_v], rows_v, sem).wait()
        pltpu.sync_copy(dst_hbm.at[pl.ds(off, CH1)], idx_v)
        pltpu.sync_copy(rows_v, acc_sh.at[idx_v], add=True)
        pltpu.sync_copy(ones_v.at[pl.ds(0, CH1)], cnt_sh.at[idx_v], add=True)
        return carry

    lax.fori_loop(0, NCH1, chunk, 0)
    plsc.subcore_barrier()

    pltpu.sync_copy(acc_sh.at[pl.ds(nbase, SLICE1)], zbuf_v)
    pltpu.sync_copy(zbuf_v, acc_out.at[c, pl.ds(nbase, SLICE1)])
    pltpu.sync_copy(cnt_sh.at[pl.ds(nbase, SLICE1)], zcnt_v.at[pl.ds(0, SLICE1)])
    pltpu.sync_copy(zcnt_v.at[pl.ds(0, SLICE1)], cnt_out.at[c, pl.ds(nbase, SLICE1)])


_sc1 = pl.kernel(
    _sc1_body,
    out_type=(
        jax.ShapeDtypeStruct((NC, NPAD1, H), jnp.float32),
        jax.ShapeDtypeStruct((NC, NPAD1), jnp.float32),
    ),
    mesh=_mesh,
    scratch_types=[
        pltpu.VMEM((CH1,), jnp.int32),
        pltpu.VMEM((CH1, H), jnp.float32),
        pltpu.VMEM((CH1 + 16,), jnp.float32),
        pltpu.VMEM((SLICE1, H), jnp.float32),
        pltpu.VMEM((SLICE1,), jnp.float32),
        pltpu.VMEM_SHARED((NPAD1, H), jnp.float32),
        pltpu.VMEM_SHARED((NPAD1,), jnp.float32),
        pltpu.SemaphoreType.DMA,
    ],
    compiler_params=_sc_params,
)


def _sc2_body(h_hbm, src_hbm, dst_hbm, mx_out,
              dch_v, sch_v, stag_v, csr_v, cnt16_v, pub_v,
              ent_v, srcl_v, rows_v, acc_v,
              cnts_sm, offs_sm, bins_sh, counts_sh, offsets_sh, sem):
    c = lax.axis_index("c")
    s = lax.axis_index("s")
    lanei = lax.iota(jnp.int32, 16)

    r_cal, _ = plsc.scan_count(jnp.zeros((16,), jnp.int32))
    base = jnp.min(r_cal)

    cnt16_v[pl.ds(0, 16)] = jnp.zeros((16,), jnp.int32)
    ebase = s * EPS

    def a1_chunk(k, carry):
        off = pl.multiple_of(ebase + k * CH2, 8)
        pltpu.sync_copy(dst_hbm.at[pl.ds(off, CH2)], dch_v)
        pltpu.sync_copy(src_hbm.at[pl.ds(off, CH2)], sch_v)

        def a1_vec(j, carry2):
            d16 = dch_v[pl.ds(j * 16, 16)]
            s16 = sch_v[pl.ds(j * 16, 16)]
            keep = (d16 & 1) == c
            o16 = (d16 >> 1) & 15
            l16 = d16 >> 5
            osent = jnp.where(keep, o16, 31)
            packed = (osent << 23) | (s16 << 9) | l16
            stag_v[pl.ds(k * CH2 + j * 16, 16)] = packed
            plsc.addupdate_scatter(
                cnt16_v, [o16], jnp.where(keep, 1, 0).astype(jnp.int32))
            return carry2

        lax.fori_loop(0, CH2 // 16, a1_vec, 0)
        return carry

    lax.fori_loop(0, NCH2, a1_chunk, 0)

    cnt16 = cnt16_v[pl.ds(0, 16)]
    padded = (cnt16 + 7) & (-8)
    incl = plsc.cumsum(padded)
    excl = incl - padded
    cnt16_v[pl.ds(0, 16)] = excl
    pub_v[pl.ds(0, 16)] = cnt16
    pub_v[pl.ds(16, 16)] = excl
    pltpu.sync_copy(pub_v.at[pl.ds(0, 16)], counts_sh.at[s])
    pltpu.sync_copy(pub_v.at[pl.ds(16, 16)], offsets_sh.at[s])

    def a2_vec(i, carry):
        p16 = stag_v[pl.ds(i * 16, 16)]
        o16 = (p16 >> 23) & 31
        sk, sv = plsc.sort_key_val(o16, p16)
        kp = sk < 16
        r16, lm = plsc.scan_count(sk)
        rank0 = r16 - base
        ok = jnp.minimum(sk, 15)
        pos = plsc.load_gather(cnt16_v, [ok]) + rank0
        posc = jnp.clip(pos, 0, CSRCAP - 1)
        plsc.store_scatter(csr_v, [posc], sv & 0x7FFFFF, mask=kp)
        plsc.store_scatter(cnt16_v, [ok], posc + 1, mask=lm & kp)
        return carry

    lax.fori_loop(0, EPS // 16, a2_vec, 0)

    pltpu.sync_copy(csr_v, bins_sh.at[s, pl.ds(0, CSRCAP)])
    plsc.subcore_barrier()

    _zero_rows(acc_v, LRUN)
    pltpu.sync_copy(counts_sh, cnts_sm)
    pltpu.sync_copy(offsets_sh, offs_sm)

    def w_body(w, carry):
        run_len = cnts_sm[w, s]
        run_st = offs_sm[w, s]
        nch = (run_len + CHB - 1) // CHB

        def ch_body(q, carry2):
            off = pl.multiple_of(run_st + q * CHB, 8)
            pltpu.sync_copy(bins_sh.at[w, pl.ds(off, CHB)], ent_v)

            def sl_body(g, carry3):
                e16 = ent_v[pl.ds(g * 16, 16)]
                srcl_v[pl.ds(g * 16, 16)] = jnp.minimum((e16 >> 9) & 0x3FFF, N - 1)
                return carry3

            lax.fori_loop(0, CHB // 16, sl_body, 0)
            pltpu.async_copy(h_hbm.at[srcl_v], rows_v, sem).wait()

            v = jnp.minimum(run_len - q * CHB, CHB)
            ng = (v + 15) // 16

            def g_body(g, carry3):
                e16 = ent_v[pl.ds(g * 16, 16)]
                l16 = e16 & 511
                valid = (g * 16 + lanei) < v
                lkey = jnp.where(valid, l16, 1023)
                sk, sv = plsc.sort_key_val(lkey, lanei)
                r16, _lm = plsc.scan_count(sk)
                rank0 = r16 - base
                vs = sk < 1023
                maxr = jnp.max(jnp.where(vs, rank0, 0))
                lcl = jnp.minimum(sk, LRUN - 1)
                rowi = g * 16 + sv

                def p_body(r, carry4):
                    m = vs & (rank0 == r)
                    for h in range(H):
                        hh = jnp.full((16,), h, jnp.int32)
                        a = plsc.load_gather(acc_v, [lcl, hh], mask=m)
                        mv = plsc.load_gather(rows_v, [rowi, hh], mask=m)
                        plsc.store_scatter(acc_v, [lcl, hh], jnp.maximum(a, mv),
                                           mask=m)
                    return carry4

                lax.fori_loop(0, maxr + 1, p_body, 0)
                return carry3

            lax.fori_loop(0, ng, g_body, 0)
            return carry2

        lax.fori_loop(0, nch, ch_body, 0)
        return carry

    lax.fori_loop(0, NS, w_body, 0)

    pltpu.sync_copy(acc_v, mx_out.at[c, s])


_sc2 = pl.kernel(
    _sc2_body,
    out_type=jax.ShapeDtypeStruct((NC, NS, LRUN, H), jnp.float32),
    mesh=_mesh,
    scratch_types=[
        pltpu.VMEM((CH2,), jnp.int32),
        pltpu.VMEM((CH2,), jnp.int32),
        pltpu.VMEM((EPS,), jnp.int32),
        pltpu.VMEM((CSRCAP,), jnp.int32),
        pltpu.VMEM((16,), jnp.int32),
        pltpu.VMEM((32,), jnp.int32),
        pltpu.VMEM((CHB,), jnp.int32),
        pltpu.VMEM((CHB,), jnp.int32),
        pltpu.VMEM((CHB, H), jnp.float32),
        pltpu.VMEM((LRUN, H), jnp.float32),
        pltpu.SMEM((NS, NS), jnp.int32),
        pltpu.SMEM((NS, NS), jnp.int32),
        pltpu.VMEM_SHARED((NS, BINROW), jnp.int32),
        pltpu.VMEM_SHARED((NS, NS), jnp.int32),
        pltpu.VMEM_SHARED((NS, NS), jnp.int32),
        pltpu.SemaphoreType.DMA,
    ],
    compiler_params=_sc_params,
)


_BLK = 2000


def _tca_body(x_ref, w_ref, o_ref):
    o_ref[...] = lax.dot_general(
        x_ref[...], w_ref[...], (((1,), (1,)), ((), ())),
        preferred_element_type=jnp.float32)


def _tc_a(x, W1):
    return pl.pallas_call(
        _tca_body,
        grid=(N // _BLK,),
        in_specs=[
            pl.BlockSpec((_BLK, D), lambda i: (i, 0)),
            pl.BlockSpec((H, D), lambda i: (0, 0)),
        ],
        out_specs=pl.BlockSpec((_BLK, H), lambda i: (i, 0)),
        out_shape=jax.ShapeDtypeStruct((N, H), jnp.float32),
    )(x, W1)


def _tcb_body(acc_ref, cnt_ref, b1_ref, o_ref):
    a = acc_ref[0] + acc_ref[1]
    cnt = cnt_ref[0] + cnt_ref[1]
    cnt = jnp.maximum(cnt, 1.0)
    o_ref[...] = jnp.maximum(a / cnt + b1_ref[...][None, :], 0.0)


def _tc_b(acc, cnt, b1):
    return pl.pallas_call(
        _tcb_body,
        grid=(N // _BLK,),
        in_specs=[
            pl.BlockSpec((NC, _BLK, H), lambda i: (0, i, 0)),
            pl.BlockSpec((NC, _BLK, 1), lambda i: (0, i, 0)),
            pl.BlockSpec((H,), lambda i: (0,)),
        ],
        out_specs=pl.BlockSpec((_BLK, H), lambda i: (i, 0)),
        out_shape=jax.ShapeDtypeStruct((N, H), jnp.float32),
    )(acc, cnt, b1)


def _tcc_body(mx_ref, w_ref, b2_ref, o_ref):
    logits = lax.dot_general(
        mx_ref[...], w_ref[...], (((1,), (1,)), ((), ())),
        preferred_element_type=jnp.float32) + b2_ref[...][None, :]
    m = jnp.max(logits, axis=1, keepdims=True)
    z = logits - m
    lse = jnp.log(jnp.sum(jnp.exp(z), axis=1, keepdims=True))
    o_ref[...] = z - lse


def _tc_c(mx, W2, b2):
    return pl.pallas_call(
        _tcc_body,
        grid=(N // _BLK,),
        in_specs=[
            pl.BlockSpec((_BLK, H), lambda i: (i, 0)),
            pl.BlockSpec((C, H), lambda i: (0, 0)),
            pl.BlockSpec((C,), lambda i: (0,)),
        ],
        out_specs=pl.BlockSpec((_BLK, C), lambda i: (i, 0)),
        out_shape=jax.ShapeDtypeStruct((N, C), jnp.float32),
    )(mx, W2, b2)


def kernel(x, edge_index, W1, b1, W2, b2):
    src = edge_index[0]
    dst = edge_index[1]

    y = _tc_a(x, W1)
    acc, cnt = _sc1(y, src, dst)
    h = _tc_b(acc[:, :N], cnt[:, :N, None], b1)
    mx4 = _sc2(h, src, dst)
    mx = mx4.transpose(2, 1, 0, 3).reshape(NC * NS * LRUN, H)[:N]
    return _tc_c(mx, W2, b2)


# speedup vs baseline: 2.6173x; 2.6173x over previous
import functools

import jax
import jax.numpy as jnp
from jax import lax
from jax.experimental import pallas as pl
from jax.experimental.pallas import tpu as pltpu
from jax.experimental.pallas import tpu_sc as plsc

N = 10000
E = 320000
D = 128
H = 32
C = 16

NC = 2
NS = 16
NW = NC * NS

SLICE1 = 640
NPAD1 = NS * SLICE1
EPT = E // NW
CH = 400
NCHK = EPT // CH
NOWN = 32
LRUN = 320
CSR2 = EPT + NOWN * 16
BINPAD = CSR2 + 1024
SENT = 511

_mesh = plsc.VectorSubcoreMesh(core_axis_name="c", subcore_axis_name="s")
_sc_params = pltpu.CompilerParams(use_tc_tiling_on_sc=False,
                                  needs_layout_passes=False)


def _fill_f32(ref, n_vecs, val):
    v = jnp.full((16,), val, jnp.float32)

    def body(i, carry):
        ref[pl.ds(i * 16, 16)] = v
        return carry

    lax.fori_loop(0, n_vecs, body, 0)


def _fill_i32(ref, n_vecs, val):
    v = jnp.full((16,), val, jnp.int32)

    def body(i, carry):
        ref[pl.ds(i * 16, 16)] = v
        return carry

    lax.fori_loop(0, n_vecs, body, 0)


def _zero_rows(ref, rows):
    z = jnp.zeros((16,), jnp.float32)

    def body(i, carry):
        r = i >> 1
        cs = (i & 1) * 16
        ref[r, pl.ds(cs, 16)] = z
        return carry

    lax.fori_loop(0, rows * 2, body, 0)


def _scan_base():
    r_cal, _ = plsc.scan_count(jnp.zeros((16,), jnp.int32))
    return jnp.min(r_cal)


def _sc1_body(y_hbm, src_hbm, dst_hbm, acc_out, cnt_out, bins_out, meta_out,
              src_v, dst_v, rows_v, src2_v, dst2_v, rows2_v,
              ones_v, zbuf_v, zcnt_v, stag_v, csr_v,
              cnt32_v, pub_v, acc_sh, cnt_sh, sem, sem2):
    c = lax.axis_index("c")
    s = lax.axis_index("s")
    wid = c * NS + s
    base = _scan_base()

    _zero_rows(zbuf_v, LRUN)
    _fill_f32(zcnt_v, SLICE1 // 16, 0.0)
    _fill_f32(ones_v, CH // 16, 1.0)
    _fill_i32(csr_v, CSR2 // 16, SENT)
    cnt32_v[pl.ds(0, 16)] = jnp.zeros((16,), jnp.int32)
    cnt32_v[pl.ds(16, 16)] = jnp.zeros((16,), jnp.int32)
    nbase = s * SLICE1
    pltpu.sync_copy(zbuf_v, acc_sh.at[pl.ds(nbase, LRUN)])
    pltpu.sync_copy(zbuf_v, acc_sh.at[pl.ds(nbase + LRUN, LRUN)])
    pltpu.sync_copy(zcnt_v, cnt_sh.at[pl.ds(nbase, SLICE1)])
    plsc.subcore_barrier()

    ebase = wid * EPT
    bufs = [(src_v, dst_v, rows_v, sem), (src2_v, dst2_v, rows2_v, sem2)]

    def _load(k):
        sv, dv, rv, sm = bufs[k % 2]
        off = pl.multiple_of(ebase + k * CH, 8)
        pltpu.sync_copy(src_hbm.at[pl.ds(off, CH)], sv)
        gat = pltpu.async_copy(y_hbm.at[sv], rv, sem=sm)
        pltpu.sync_copy(dst_hbm.at[pl.ds(off, CH)], dv)
        return gat

    def _bin(k):
        sv, dv, _rv, _sm = bufs[k % 2]

        def binvec(j, carry2):
            d16 = dv[pl.ds(j * 16, 16)]
            s16 = sv[pl.ds(j * 16, 16)]
            o16 = d16 & 31
            l16 = d16 >> 5
            stag_v[pl.ds(k * CH + j * 16, 16)] = (o16 << 23) | (s16 << 9) | l16
            plsc.addupdate_scatter(cnt32_v, [o16], jnp.ones((16,), jnp.int32))
            return carry2

        lax.fori_loop(0, CH // 16, binvec, 0)

    def _scatter(k, gat):
        _sv, dv, rv, _sm = bufs[k % 2]
        gat.wait()
        pltpu.sync_copy(rv, acc_sh.at[dv], add=True)
        pltpu.sync_copy(ones_v, cnt_sh.at[dv], add=True)

    gats = {0: _load(0)}
    for k in range(NCHK):
        if k + 1 < NCHK:
            gats[k + 1] = _load(k + 1)
        _bin(k)
        _scatter(k, gats.pop(k))

    c0 = cnt32_v[pl.ds(0, 16)]
    c1 = cnt32_v[pl.ds(16, 16)]
    p0 = (c0 + 15) & (-16)
    p1 = (c1 + 15) & (-16)
    i0 = plsc.cumsum(p0)
    i1 = plsc.cumsum(p1) + jnp.max(i0)
    e0 = i0 - p0
    e1 = i1 - p1
    pub_v[pl.ds(0, 16)] = c0
    pub_v[pl.ds(16, 16)] = c1
    pub_v[pl.ds(32, 16)] = e0
    pub_v[pl.ds(48, 16)] = e1
    cnt32_v[pl.ds(0, 16)] = e0
    cnt32_v[pl.ds(16, 16)] = e1
    pltpu.sync_copy(pub_v, meta_out.at[wid])

    def _place_prolog(i):
        p16 = stag_v[pl.ds(i * 16, 16)]
        o16 = (p16 >> 23) & 31
        sk, sv = plsc.sort_key_val(o16, p16)
        r16, lm = plsc.scan_count(sk)
        return sk, sv, r16 - base, lm

    def _place_commit(sk, sv, rank0, lm):
        pos = plsc.load_gather(cnt32_v, [sk]) + rank0
        posc = jnp.clip(pos, 0, CSR2 - 1)
        plsc.store_scatter(csr_v, [posc], sv & 0x7FFFFF)
        plsc.store_scatter(cnt32_v, [sk], posc + 1, mask=lm)

    def place2(i, carry):
        a = _place_prolog(2 * i)
        b = _place_prolog(2 * i + 1)
        _place_commit(*a)
        _place_commit(*b)
        return carry

    lax.fori_loop(0, EPT // 32, place2, 0)
    if EPT // 16 % 2:
        _place_commit(*_place_prolog(EPT // 16 - 1))
    pltpu.sync_copy(csr_v, bins_out.at[wid, pl.ds(0, CSR2)])

    plsc.subcore_barrier()
    pltpu.sync_copy(acc_sh.at[pl.ds(nbase, LRUN)], zbuf_v)
    pltpu.sync_copy(zbuf_v, acc_out.at[c, pl.ds(nbase, LRUN)])
    pltpu.sync_copy(acc_sh.at[pl.ds(nbase + LRUN, LRUN)], zbuf_v)
    pltpu.sync_copy(zbuf_v, acc_out.at[c, pl.ds(nbase + LRUN, LRUN)])
    pltpu.sync_copy(cnt_sh.at[pl.ds(nbase, SLICE1)], zcnt_v)
    pltpu.sync_copy(zcnt_v, cnt_out.at[c, pl.ds(nbase, SLICE1)])


_sc1 = pl.kernel(
    _sc1_body,
    out_type=(
        jax.ShapeDtypeStruct((NC, NPAD1, H), jnp.float32),
        jax.ShapeDtypeStruct((NC, NPAD1), jnp.float32),
        jax.ShapeDtypeStruct((NW, BINPAD), jnp.int32),
        jax.ShapeDtypeStruct((NW, 64), jnp.int32),
    ),
    mesh=_mesh,
    scratch_types=[
        pltpu.VMEM((CH,), jnp.int32),
        pltpu.VMEM((CH,), jnp.int32),
        pltpu.VMEM((CH, H), jnp.float32),
        pltpu.VMEM((CH,), jnp.int32),
        pltpu.VMEM((CH,), jnp.int32),
        pltpu.VMEM((CH, H), jnp.float32),
        pltpu.VMEM((CH,), jnp.float32),
        pltpu.VMEM((LRUN, H), jnp.float32),
        pltpu.VMEM((SLICE1,), jnp.float32),
        pltpu.VMEM((EPT,), jnp.int32),
        pltpu.VMEM((CSR2,), jnp.int32),
        pltpu.VMEM((32,), jnp.int32),
        pltpu.VMEM((64,), jnp.int32),
        pltpu.VMEM_SHARED((NPAD1, H), jnp.float32),
        pltpu.VMEM_SHARED((NPAD1,), jnp.float32),
        pltpu.SemaphoreType.DMA,
        pltpu.SemaphoreType.DMA,
    ],
    compiler_params=_sc_params,
)


def _sc2_body(h_hbm, bins_hbm, meta_hbm, mx_out,
              meta_v, ent_v, srcl_v, rows_v, acc_v, sem, sem2):
    c = lax.axis_index("c")
    s = lax.axis_index("s")
    o = 2 * s + c

    _zero_rows(acc_v, LRUN)
    pltpu.sync_copy(meta_hbm, meta_v)

    def build_and_rmw(ng):
        def slb(g, carry):
            e16 = ent_v[pl.ds(g * 16, 16)]
            srcl_v[pl.ds(g * 16, 16)] = jnp.minimum((e16 >> 9) & 0x3FFF, N - 1)
            return carry

        lax.fori_loop(0, 64, slb, 0)
        ga = pltpu.async_copy(h_hbm.at[srcl_v.at[pl.ds(0, 512)]],
                              rows_v.at[pl.ds(0, 512)], sem)
        gb2 = pltpu.async_copy(h_hbm.at[srcl_v.at[pl.ds(512, 512)]],
                               rows_v.at[pl.ds(512, 512)], sem2)

        def gb(g, carry):
            e16 = ent_v[pl.ds(g * 16, 16)]
            l16 = jnp.minimum(e16 & 511, LRUN - 1)
            for j in range(16):
                lj = l16[j]
                r = g * 16 + j
                a0 = acc_v[lj, pl.ds(0, 16)]
                m0 = rows_v[r, pl.ds(0, 16)]
                acc_v[lj, pl.ds(0, 16)] = jnp.maximum(a0, m0)
                a1 = acc_v[lj, pl.ds(16, 16)]
                m1 = rows_v[r, pl.ds(16, 16)]
                acc_v[lj, pl.ds(16, 16)] = jnp.maximum(a1, m1)
            return carry

        ga.wait()
        lax.fori_loop(0, jnp.minimum(ng, 32), gb, 0)
        gb2.wait()
        lax.fori_loop(32, jnp.maximum(ng, 32), gb, 0)

    def w_body(w, f):
        w16 = jnp.full((16,), 0, jnp.int32) + w
        o16 = jnp.full((16,), o, jnp.int32)
        rl = jnp.max(plsc.load_gather(meta_v, [w16, o16]))
        st = jnp.max(plsc.load_gather(meta_v, [w16, o16 + 32]))
        rl16 = (rl + 15) & (-16)

        def app_cond(state):
            pos, _f = state
            return pos < rl16

        def app_body(state):
            pos, f1 = state
            srcoff = pl.multiple_of(st + pos, 8)
            pltpu.sync_copy(bins_hbm.at[w, pl.ds(srcoff, 1024)],
                            ent_v.at[pl.ds(pl.multiple_of(f1, 8), 1024)])
            t = jnp.minimum(rl16 - pos, 1024)
            f2 = f1 + t

            def flush_cond(fx):
                return fx >= 1024

            def flush_body(fx):
                build_and_rmw(64)

                def sh(i, carry):
                    ent_v[pl.ds(i * 16, 16)] = ent_v[pl.ds(1024 + i * 16, 16)]
                    return carry

                lax.fori_loop(0, 128, sh, 0)
                return fx - 1024

            f3 = lax.while_loop(flush_cond, flush_body, f2)
            return (pos + t, f3)

        _pos, f_out = lax.while_loop(app_cond, app_body, (jnp.int32(0), f))
        return f_out

    f_final = lax.fori_loop(0, NW, w_body, jnp.int32(0))
    build_and_rmw(f_final // 16)

    pltpu.sync_copy(acc_v, mx_out.at[c, s])


_sc2 = pl.kernel(
    _sc2_body,
    out_type=jax.ShapeDtypeStruct((NC, NS, LRUN, H), jnp.float32),
    mesh=_mesh,
    scratch_types=[
        pltpu.VMEM((NW, 64), jnp.int32),
        pltpu.VMEM((3072,), jnp.int32),
        pltpu.VMEM((1024,), jnp.int32),
        pltpu.VMEM((1024, H), jnp.float32),
        pltpu.VMEM((LRUN, H), jnp.float32),
        pltpu.SemaphoreType.DMA,
        pltpu.SemaphoreType.DMA,
    ],
    compiler_params=_sc_params,
)


_BLK = 2000


def _tca_body(x_ref, w_ref, o_ref):
    o_ref[...] = lax.dot_general(
        x_ref[...], w_ref[...], (((1,), (1,)), ((), ())),
        preferred_element_type=jnp.float32)


def _tc_a(x, W1):
    return pl.pallas_call(
        _tca_body,
        grid=(N // _BLK,),
        in_specs=[
            pl.BlockSpec((_BLK, D), lambda i: (i, 0)),
            pl.BlockSpec((H, D), lambda i: (0, 0)),
        ],
        out_specs=pl.BlockSpec((_BLK, H), lambda i: (i, 0)),
        out_shape=jax.ShapeDtypeStruct((N, H), jnp.float32),
    )(x, W1)


def _tcb_body(acc_ref, cnt_ref, b1_ref, o_ref):
    a = acc_ref[0] + acc_ref[1]
    cnt = cnt_ref[0] + cnt_ref[1]
    cnt = jnp.maximum(cnt, 1.0)
    o_ref[...] = jnp.maximum(a / cnt + b1_ref[...][None, :], 0.0)


def _tc_b(acc, cnt, b1):
    return pl.pallas_call(
        _tcb_body,
        grid=(N // _BLK,),
        in_specs=[
            pl.BlockSpec((NC, _BLK, H), lambda i: (0, i, 0)),
            pl.BlockSpec((NC, _BLK, 1), lambda i: (0, i, 0)),
            pl.BlockSpec((H,), lambda i: (0,)),
        ],
        out_specs=pl.BlockSpec((_BLK, H), lambda i: (i, 0)),
        out_shape=jax.ShapeDtypeStruct((N, H), jnp.float32),
    )(acc, cnt, b1)


def _tcc_body(mx_ref, w_ref, b2_ref, o_ref):
    logits = lax.dot_general(
        mx_ref[...], w_ref[...], (((1,), (1,)), ((), ())),
        preferred_element_type=jnp.float32) + b2_ref[...][None, :]
    m = jnp.max(logits, axis=1, keepdims=True)
    z = logits - m
    lse = jnp.log(jnp.sum(jnp.exp(z), axis=1, keepdims=True))
    o_ref[...] = z - lse


def _tc_c(mx, W2, b2):
    return pl.pallas_call(
        _tcc_body,
        grid=(N // _BLK,),
        in_specs=[
            pl.BlockSpec((_BLK, H), lambda i: (i, 0)),
            pl.BlockSpec((C, H), lambda i: (0, 0)),
            pl.BlockSpec((C,), lambda i: (0,)),
        ],
        out_specs=pl.BlockSpec((_BLK, C), lambda i: (i, 0)),
        out_shape=jax.ShapeDtypeStruct((N, C), jnp.float32),
    )(mx, W2, b2)


def kernel(x, edge_index, W1, b1, W2, b2):
    src = edge_index[0]
    dst = edge_index[1]

    y = _tc_a(x, W1)
    acc, cnt, bins, meta = _sc1(y, src, dst)
    h = _tc_b(acc[:, :N], cnt[:, :N, None], b1)
    mx4 = _sc2(h, bins, meta)
    mx = mx4.transpose(2, 1, 0, 3).reshape(NC * NS * LRUN, H)[:N]
    return _tc_c(mx, W2, b2)
